# baseline (speedup 1.0000x reference)
import functools

import jax
import jax.numpy as jnp
from jax import lax
from jax.experimental import pallas as pl
from jax.experimental.pallas import tpu as pltpu
from jax.experimental.pallas import tpu_sc as plsc

_B, _H, _W, _C = 64, 128, 128, 4
_HW = _H * _W
_XW = _HW * _C
_L = 16
_GROUPS = _HW // _L
_GPR = _W // _L
_BIG = 1 << 30

_NC, _NS = 2, 16
_NW = _NC * _NS
_SPT = _B // _NW

_mesh = plsc.VectorSubcoreMesh(core_axis_name="c", subcore_axis_name="s",
                               num_cores=_NC, num_subcores=_NS)


def _oracle_body(x_hbm, out1, out2, xbuf, zbuf):
    wid = lax.axis_index("s") * _NC + lax.axis_index("c")
    iota = lax.iota(jnp.int32, _L)
    zeros_v = jnp.zeros((_L,), jnp.float32)
    big_v = jnp.full((_L,), _BIG, jnp.int32)
    lane0 = iota == 0
    idx3_0 = iota * 4 + 3
    idx1 = [iota * 4 + 1 + 64 * j for j in range(_GPR)]

    def zero_body(g, carry):
        zbuf[pl.ds(g * _L, _L)] = zeros_v
        return carry

    lax.fori_loop(0, _GROUPS, zero_body, 0)

    for s in range(_SPT):
        b = wid * _SPT + s
        pltpu.sync_copy(x_hbm.at[b], xbuf)
        pltpu.sync_copy(zbuf, out2.at[b])

        def p1_body(g, acc):
            v = plsc.load_gather(xbuf, [idx3_0 + g * (4 * _L)])
            cand = jnp.where(v == 1.0, iota + g * _L, _BIG)
            return jnp.minimum(acc, cand)

        acc1 = lax.fori_loop(0, _GROUPS, p1_body, big_v)
        opp_min = jnp.min(acc1)
        has_opp = opp_min < _BIG
        opp_flat = jnp.where(has_opp, opp_min, 0)
        opp_r = opp_flat >> 7
        opp_c = opp_flat & (_W - 1)

        oc_v = jnp.broadcast_to(opp_c, (_L,))
        keys_j = []
        for j in range(_GPR):
            dc = (iota + 16 * j) - oc_v
            keys_j.append(dc * dc * _HW + (iota + 16 * j))

        def p2_row(r, carry):
            acc, cnt = carry
            dr = r - opp_r
            s_v = jnp.broadcast_to(dr * dr * _HW + r * _W,
                                   (_L,))
            rb_v = jnp.broadcast_to(r * (4 * _W), (_L,))
            for j in range(_GPR):
                v = plsc.load_gather(xbuf, [rb_v + idx1[j]])
                m = v == 1.0
                acc = jnp.minimum(acc, jnp.where(m, s_v + keys_j[j], _BIG))
                cnt = cnt + m.astype(jnp.int32)
            return acc, cnt

        acc2, cnt = lax.fori_loop(0, _H, p2_row,
                                  (big_v, jnp.zeros((_L,), jnp.int32)))
        fkey = jnp.min(acc2)
        n_food = jnp.sum(cnt)

        tgt = jnp.where(fkey < _BIG, fkey & (_HW - 1), 0)
        opp_is_start = has_opp & (opp_flat == 3 * _W + 6)
        use_argmin = ((n_food > 1) & has_opp & (~opp_is_start)) | (n_food == 1)
        target = jnp.where(use_argmin, tgt, 0)
        val = jnp.where(n_food > 0, jnp.float32(1.0), jnp.float32(0.0))

        t_v = jnp.broadcast_to(target, (_L,))
        plsc.store_scatter(zbuf, [t_v], jnp.broadcast_to(val, (_L,)),
                           mask=lane0)
        pltpu.sync_copy(zbuf, out1.at[b])
        plsc.store_scatter(zbuf, [t_v], zeros_v, mask=lane0)


_oracle = pl.kernel(
    _oracle_body,
    out_type=[jax.ShapeDtypeStruct((_B, _HW), jnp.float32),
              jax.ShapeDtypeStruct((_B, _HW), jnp.float32)],
    mesh=_mesh,
    scratch_types=[pltpu.VMEM((_XW,), jnp.float32),
                   pltpu.VMEM((_HW,), jnp.float32)],
    compiler_params=pltpu.CompilerParams(needs_layout_passes=False),
)


@jax.jit
def kernel(x, history):
    del history
    out1, out2 = _oracle(x.reshape(_B, _XW))
    return out1.reshape(_B, _H, _W), out2.reshape(_B, _H, _W)


# speedup vs baseline: 3.0010x; 3.0010x over previous
import jax
import jax.numpy as jnp
from jax import lax
from jax.experimental import pallas as pl
from jax.experimental.pallas import tpu as pltpu
from jax.experimental.pallas import tpu_sc as plsc

_B, _H, _W, _C = 64, 128, 128, 4
_HW = _H * _W
_L = 16
_GPR = _W // _L
_BIG = 1 << 30

_NC, _NS = 2, 16
_NW = _NC * _NS
_SPT = _B // _NW

_mesh = plsc.VectorSubcoreMesh(core_axis_name="c", subcore_axis_name="s",
                               num_cores=_NC, num_subcores=_NS)


def _oracle_body(x_hbm, out1, out2,
                 xb3a, xb1a, xb3b, xb1b, zbuf,
                 i3a, i1a, i3b, i1b,
                 s3a, s1a, s3b, s1b, so2a, so2b, so1):
    wid = lax.axis_index("s") * _NC + lax.axis_index("c")
    iota = lax.iota(jnp.int32, _L)
    zeros_v = jnp.zeros((_L,), jnp.float32)
    big_v = jnp.full((_L,), _BIG, jnp.int32)
    lane0 = iota == 0
    cvecs = [iota + 16 * j for j in range(_GPR)]

    b0 = wid * _SPT
    b1 = b0 + 1

    def write_idx(ref, b, ch):
        base = b * (_H * _C) + ch
        for g in range(_GPR):
            ref[pl.ds(16 * g, _L)] = base + 4 * (iota + 16 * g)

    write_idx(i3a, b0, 3)
    write_idx(i1a, b0, 1)
    write_idx(i3b, b1, 3)
    write_idx(i1b, b1, 1)
    c3a = pltpu.async_copy(x_hbm.at[i3a], xb3a, s3a)
    c1a = pltpu.async_copy(x_hbm.at[i1a], xb1a, s1a)
    c3b = pltpu.async_copy(x_hbm.at[i3b], xb3b, s3b)
    c1b = pltpu.async_copy(x_hbm.at[i1b], xb1b, s1b)

    def zero_body(r, carry):
        for j in range(_GPR):
            zbuf[r, pl.ds(16 * j, _L)] = zeros_v
        return carry

    lax.fori_loop(0, _H, zero_body, 0)
    o2a = pltpu.async_copy(zbuf, out2.at[b0], so2a)
    o2b = pltpu.async_copy(zbuf, out2.at[b1], so2b)

    def phase1(buf):
        def p1_body(r, acc):
            rb_v = jnp.broadcast_to(r * _W, (_L,))
            for j in range(_GPR):
                v = buf[r, pl.ds(16 * j, _L)]
                cand = jnp.where(v == 1.0, rb_v + cvecs[j], _BIG)
                acc = jnp.minimum(acc, cand)
            return acc

        acc1 = lax.fori_loop(0, _H, p1_body, big_v)
        return jnp.min(acc1)

    def phase2(buf, opp_min):
        has_opp = opp_min < _BIG
        opp_flat = jnp.where(has_opp, opp_min, 0)
        opp_r = opp_flat >> 7
        opp_c = opp_flat & (_W - 1)
        oc_v = jnp.broadcast_to(opp_c, (_L,))
        keys_j = []
        for j in range(_GPR):
            dc = cvecs[j] - oc_v
            keys_j.append(dc * dc * _HW + cvecs[j])

        def p2_body(r, carry):
            acc, cnt = carry
            dr = r - opp_r
            s_v = jnp.broadcast_to(dr * dr * _HW + r * _W, (_L,))
            for j in range(_GPR):
                v = buf[r, pl.ds(16 * j, _L)]
                m = v == 1.0
                acc = jnp.minimum(acc, jnp.where(m, s_v + keys_j[j], _BIG))
                cnt = cnt + m.astype(jnp.int32)
            return acc, cnt

        acc2, cnt = lax.fori_loop(0, _H, p2_body,
                                  (big_v, jnp.zeros((_L,), jnp.int32)))
        fkey = jnp.min(acc2)
        n_food = jnp.sum(cnt)

        tgt = jnp.where(fkey < _BIG, fkey & (_HW - 1), 0)
        opp_is_start = has_opp & (opp_flat == 3 * _W + 6)
        use_argmin = (((n_food > 1) & has_opp & (~opp_is_start))
                      | (n_food == 1))
        target = jnp.where(use_argmin, tgt, 0)
        val = jnp.where(n_food > 0, jnp.float32(1.0), jnp.float32(0.0))
        return target, val

    def scatter(target, val):
        tr_v = jnp.broadcast_to(target >> 7, (_L,))
        tc_v = jnp.broadcast_to(target & (_W - 1), (_L,))
        plsc.store_scatter(zbuf, [tr_v, tc_v],
                           jnp.broadcast_to(val, (_L,)), mask=lane0)
        return tr_v, tc_v

    c3a.wait()
    opp0 = phase1(xb3a)
    c1a.wait()
    target0, val0 = phase2(xb1a, opp0)
    o2a.wait()
    o2b.wait()
    tr0, tc0 = scatter(target0, val0)
    o1a = pltpu.async_copy(zbuf, out1.at[b0], so1)

    c3b.wait()
    opp1 = phase1(xb3b)
    c1b.wait()
    target1, val1 = phase2(xb1b, opp1)
    o1a.wait()
    plsc.store_scatter(zbuf, [tr0, tc0], zeros_v, mask=lane0)
    scatter(target1, val1)
    pltpu.sync_copy(zbuf, out1.at[b1])


_oracle = pl.kernel(
    _oracle_body,
    out_type=[jax.ShapeDtypeStruct((_B, _H, _W), jnp.float32),
              jax.ShapeDtypeStruct((_B, _H, _W), jnp.float32)],
    mesh=_mesh,
    scratch_types=[pltpu.VMEM((_H, _W), jnp.float32),
                   pltpu.VMEM((_H, _W), jnp.float32),
                   pltpu.VMEM((_H, _W), jnp.float32),
                   pltpu.VMEM((_H, _W), jnp.float32),
                   pltpu.VMEM((_H, _W), jnp.float32),
                   pltpu.VMEM((_H,), jnp.int32),
                   pltpu.VMEM((_H,), jnp.int32),
                   pltpu.VMEM((_H,), jnp.int32),
                   pltpu.VMEM((_H,), jnp.int32),
                   pltpu.SemaphoreType.DMA,
                   pltpu.SemaphoreType.DMA,
                   pltpu.SemaphoreType.DMA,
                   pltpu.SemaphoreType.DMA,
                   pltpu.SemaphoreType.DMA,
                   pltpu.SemaphoreType.DMA,
                   pltpu.SemaphoreType.DMA],
    compiler_params=pltpu.CompilerParams(needs_layout_passes=False),
)


@jax.jit
def kernel(x, history):
    del history
    x_t = jnp.transpose(x, (0, 1, 3, 2)).reshape(_B * _H * _C, _W)
    out1, out2 = _oracle(x_t)
    return out1, out2
